# baseline (speedup 1.0000x reference)
import jax
import jax.numpy as jnp
from jax import lax
from jax.experimental import pallas as pl
from jax.experimental.pallas import tpu as pltpu
from jax.experimental.pallas import tpu_sc as plsc

NC = 2
NS = 16
NW = NC * NS
LANES = 128


def kernel(input_batch, table):
    batch, seq = input_batch.shape
    d_model = table.shape[1]
    tpad = jnp.pad(table, ((0, 0), (0, LANES - d_model)))

    bw = batch // NW
    assert bw * NW == batch and bw % 16 == 0 and seq % 2 == 0

    mesh = plsc.VectorSubcoreMesh(core_axis_name="c", subcore_axis_name="s")

    @pl.kernel(
        mesh=mesh,
        out_type=jax.ShapeDtypeStruct((seq, d_model, batch), table.dtype),
        compiler_params=pltpu.CompilerParams(needs_layout_passes=False),
        scratch_types=[
            pltpu.VMEM((bw, seq), jnp.int32),
            pltpu.VMEM((seq, bw), jnp.int32),
            pltpu.VMEM((bw, LANES), table.dtype),
            pltpu.VMEM((bw, LANES), table.dtype),
            pltpu.VMEM((d_model, bw), table.dtype),
            pltpu.SemaphoreType.DMA,
            pltpu.SemaphoreType.DMA,
        ],
    )
    def gather_kernel(table_hbm, idx_hbm, out_hbm,
                      idx_v, idxt_v, r0, r1, o_v, sg0, sg1):
        wid = lax.axis_index("s") * NC + lax.axis_index("c")
        b0 = wid * bw
        i16 = lax.iota(jnp.int32, 16)

        pltpu.sync_copy(idx_hbm.at[pl.ds(b0, bw)], idx_v)

        @pl.loop(0, seq)
        def _(s):
            s16 = i16 * 0 + s
            for jg in range(bw // 16):
                v = plsc.load_gather(idx_v, [i16 + 16 * jg, s16])
                idxt_v[s, pl.ds(16 * jg, 16)] = v

        def start(s, rbuf, sem):
            pltpu.async_copy(table_hbm.at[idxt_v.at[s]], rbuf, sem)

        def finish(s, rbuf, sem):
            pltpu.make_async_copy(table_hbm.at[idxt_v.at[s]], rbuf, sem).wait()
            @pl.loop(0, d_model)
            def _(d):
                d16 = i16 * 0 + d
                for jg in range(bw // 16):
                    v = plsc.load_gather(rbuf, [i16 + 16 * jg, d16])
                    o_v[d, pl.ds(16 * jg, 16)] = v
            pltpu.sync_copy(o_v,
                            out_hbm.at[s, pl.ds(0, d_model), pl.ds(b0, bw)])

        start(0, r0, sg0)
        start(1, r1, sg1)

        @pl.loop(0, seq // 2 - 1)
        def _(j):
            s0 = 2 * j
            finish(s0, r0, sg0)
            start(s0 + 2, r0, sg0)
            finish(s0 + 1, r1, sg1)
            start(s0 + 3, r1, sg1)

        finish(seq - 2, r0, sg0)
        finish(seq - 1, r1, sg1)

    out = gather_kernel(tpad, input_batch)
    return jnp.transpose(out, (2, 0, 1))


# speedup vs baseline: 1.5739x; 1.5739x over previous
import jax
import jax.numpy as jnp
from jax import lax
from jax.experimental import pallas as pl
from jax.experimental.pallas import tpu as pltpu
from jax.experimental.pallas import tpu_sc as plsc

NC = 2
NS = 16
NW = NC * NS
LANES = 128


def kernel(input_batch, table):
    batch, seq = input_batch.shape
    d_model = table.shape[1]
    t2 = jnp.pad(table, ((0, 0), (0, LANES - d_model)))
    t2 = t2.reshape(2 * table.shape[0], d_model)
    idx2 = input_batch * 2

    rows_per_w = batch // NW
    assert rows_per_w * NW == batch and rows_per_w % 2 == 0

    mesh = plsc.VectorSubcoreMesh(core_axis_name="c", subcore_axis_name="s")

    @pl.kernel(
        mesh=mesh,
        out_type=jax.ShapeDtypeStruct((batch, seq, d_model), table.dtype),
        compiler_params=pltpu.CompilerParams(use_tc_tiling_on_sc=False),
        scratch_types=[
            pltpu.VMEM((rows_per_w, seq), jnp.int32),
            pltpu.VMEM((seq, d_model), table.dtype),
            pltpu.VMEM((seq, d_model), table.dtype),
            pltpu.SemaphoreType.DMA,
            pltpu.SemaphoreType.DMA,
        ],
    )
    def gather_kernel(table_hbm, idx_hbm, out_hbm, idx_v, r0, r1, sg0, sg1):
        wid = lax.axis_index("s") * NC + lax.axis_index("c")
        base = wid * rows_per_w

        pltpu.sync_copy(idx_hbm.at[pl.ds(base, rows_per_w)], idx_v)

        pltpu.async_copy(table_hbm.at[idx_v.at[0]], r0, sg0)
        pltpu.async_copy(table_hbm.at[idx_v.at[1]], r1, sg1)

        @pl.loop(0, rows_per_w // 2 - 1)
        def _(j):
            i0 = 2 * j
            pltpu.make_async_copy(table_hbm.at[idx_v.at[i0]], r0, sg0).wait()
            pltpu.sync_copy(r0, out_hbm.at[base + i0])
            pltpu.async_copy(table_hbm.at[idx_v.at[i0 + 2]], r0, sg0)
            pltpu.make_async_copy(table_hbm.at[idx_v.at[i0 + 1]], r1, sg1).wait()
            pltpu.sync_copy(r1, out_hbm.at[base + i0 + 1])
            pltpu.async_copy(table_hbm.at[idx_v.at[i0 + 3]], r1, sg1)

        pltpu.make_async_copy(table_hbm.at[idx_v.at[rows_per_w - 2]], r0, sg0).wait()
        pltpu.sync_copy(r0, out_hbm.at[base + rows_per_w - 2])
        pltpu.make_async_copy(table_hbm.at[idx_v.at[rows_per_w - 1]], r1, sg1).wait()
        pltpu.sync_copy(r1, out_hbm.at[base + rows_per_w - 1])

    return gather_kernel(t2, idx2)


# speedup vs baseline: 1.8272x; 1.1609x over previous
import jax
import jax.numpy as jnp
from jax import lax
from jax.experimental import pallas as pl
from jax.experimental.pallas import tpu as pltpu
from jax.experimental.pallas import tpu_sc as plsc

NC, NS = 2, 16
NW = NC * NS
LANES = 128
G = 128


def kernel(input_batch, table):
    batch, seq = input_batch.shape
    num_idx = batch * seq
    d_model = table.shape[1]
    tpad = jnp.pad(table, ((0, 0), (0, LANES - d_model)))
    flat_idx = input_batch.reshape(num_idx)

    ipw = num_idx // NW
    n_steps = ipw // G

    mesh = plsc.VectorSubcoreMesh(core_axis_name="c", subcore_axis_name="s")

    @pl.kernel(
        mesh=mesh,
        out_type=jax.ShapeDtypeStruct((num_idx, LANES), table.dtype),
        scratch_types=[
            pltpu.VMEM((ipw,), jnp.int32),
            pltpu.VMEM((G, LANES), table.dtype),
            pltpu.VMEM((G, LANES), table.dtype),
            pltpu.SemaphoreType.DMA,
            pltpu.SemaphoreType.DMA,
        ],
    )
    def gather_kernel(table_hbm, idx_hbm, out_hbm, idx_v, r0, r1, sg0, sg1):
        wid = lax.axis_index("s") * NC + lax.axis_index("c")
        base = wid * ipw
        pltpu.sync_copy(idx_hbm.at[pl.ds(base, ipw)], idx_v)

        def start(g, rbuf, sem):
            pltpu.async_copy(table_hbm.at[idx_v.at[pl.ds(g * G, G)]], rbuf, sem)

        def finish(g, rbuf, sem):
            pltpu.make_async_copy(table_hbm.at[idx_v.at[pl.ds(g * G, G)]],
                                  rbuf, sem).wait()
            pltpu.sync_copy(rbuf, out_hbm.at[pl.ds(base + g * G, G)])

        start(0, r0, sg0)
        start(1, r1, sg1)

        @pl.loop(0, n_steps // 2 - 1)
        def _(j):
            g0 = 2 * j
            finish(g0, r0, sg0)
            start(g0 + 2, r0, sg0)
            finish(g0 + 1, r1, sg1)
            start(g0 + 3, r1, sg1)

        finish(n_steps - 2, r0, sg0)
        finish(n_steps - 1, r1, sg1)

    out128 = gather_kernel(tpad, flat_idx)
    return out128[:, :d_model].reshape(batch, seq, d_model)


# speedup vs baseline: 1.8294x; 1.0012x over previous
import jax
import jax.numpy as jnp
from jax import lax
from jax.experimental import pallas as pl
from jax.experimental.pallas import tpu as pltpu
from jax.experimental.pallas import tpu_sc as plsc

NC, NS = 2, 16
NW = NC * NS
LANES = 128
G = 128


def kernel(input_batch, table):
    batch, seq = input_batch.shape
    num_idx = batch * seq
    d_model = table.shape[1]
    tpad = jnp.pad(table, ((0, 0), (0, LANES - d_model)))
    flat_idx = input_batch.reshape(num_idx)

    ipw = num_idx // NW
    n_steps = ipw // G

    mesh = plsc.VectorSubcoreMesh(core_axis_name="c", subcore_axis_name="s")

    @pl.kernel(
        mesh=mesh,
        out_type=jax.ShapeDtypeStruct((num_idx, LANES), table.dtype),
        scratch_types=[
            pltpu.VMEM((ipw,), jnp.int32),
            pltpu.VMEM((G, LANES), table.dtype),
            pltpu.VMEM((G, LANES), table.dtype),
            pltpu.VMEM((G, LANES), table.dtype),
            pltpu.VMEM((G, LANES), table.dtype),
            pltpu.SemaphoreType.DMA,
            pltpu.SemaphoreType.DMA,
            pltpu.SemaphoreType.DMA,
            pltpu.SemaphoreType.DMA,
        ],
    )
    def gather_kernel(table_hbm, idx_hbm, out_hbm, idx_v,
                      r0, r1, r2, r3, sg0, sg1, sg2, sg3):
        wid = lax.axis_index("s") * NC + lax.axis_index("c")
        base = wid * ipw
        pltpu.sync_copy(idx_hbm.at[pl.ds(base, ipw)], idx_v)

        bufs = ((r0, sg0), (r1, sg1), (r2, sg2), (r3, sg3))
        nbuf = len(bufs)

        def start(g, rbuf, sem):
            pltpu.async_copy(table_hbm.at[idx_v.at[pl.ds(g * G, G)]], rbuf, sem)

        def finish(g, rbuf, sem):
            pltpu.make_async_copy(table_hbm.at[idx_v.at[pl.ds(g * G, G)]],
                                  rbuf, sem).wait()
            pltpu.sync_copy(rbuf, out_hbm.at[pl.ds(base + g * G, G)])

        for k, (rbuf, sem) in enumerate(bufs):
            start(k, rbuf, sem)

        @pl.loop(0, n_steps // nbuf - 1)
        def _(j):
            g0 = nbuf * j
            for k, (rbuf, sem) in enumerate(bufs):
                finish(g0 + k, rbuf, sem)
                start(g0 + k + nbuf, rbuf, sem)

        for k, (rbuf, sem) in enumerate(bufs):
            finish(n_steps - nbuf + k, rbuf, sem)

    out128 = gather_kernel(tpad, flat_idx)
    return out128[:, :d_model].reshape(batch, seq, d_model)
